# Initial kernel scaffold; baseline (speedup 1.0000x reference)
import jax
import jax.numpy as jnp
from jax.experimental import pallas as pl


def kernel(boxes, scores):
    raise NotImplementedError("write your pallas kernel here")


# speedup vs baseline: 16.4680x; 16.4680x over previous
import jax
import jax.numpy as jnp
from jax.experimental import pallas as pl
from jax.experimental.pallas import tpu as pltpu

_PRE_NMS_THRESH = 0.05
_PRE_NMS_TOPK = 1000
_POST_NMS_TOPK = 100
_NMS_THRESH = 0.6

_K = 1024
_B = 128
_NB = _K // _B


def _nms_kernel(rows_ref, cols_ref, out_ref, iou_ref, keep_ref):
    x0r = rows_ref[0:1, :]
    y0r = rows_ref[1:2, :]
    x1r = rows_ref[2:3, :]
    y1r = rows_ref[3:4, :]
    sr = rows_ref[4:5, :]
    area_r = jnp.maximum(x1r - x0r, 0.0) * jnp.maximum(y1r - y0r, 0.0)

    col_i = jax.lax.broadcasted_iota(jnp.int32, (1, _K), 1)
    lidx = jax.lax.broadcasted_iota(jnp.int32, (1, _B), 1)

    keep_ref[0:1, :] = jnp.where(sr > _PRE_NMS_THRESH, 1.0, 0.0)

    for b in range(_NB):
        lo = b * _B
        x0c = cols_ref[lo:lo + _B, 0:1]
        y0c = cols_ref[lo:lo + _B, 1:2]
        x1c = cols_ref[lo:lo + _B, 2:3]
        y1c = cols_ref[lo:lo + _B, 3:4]
        area_c = jnp.maximum(x1c - x0c, 0.0) * jnp.maximum(y1c - y0c, 0.0)
        w = jnp.maximum(jnp.minimum(x1c, x1r) - jnp.maximum(x0c, x0r), 0.0)
        h = jnp.maximum(jnp.minimum(y1c, y1r) - jnp.maximum(y0c, y0r), 0.0)
        inter = w * h
        union = area_c + area_r - inter
        iou_ref[:, :] = inter / jnp.maximum(union, 1e-6)

        def body(c, kb):
            rows8 = iou_ref[pl.ds(c * 8, 8), lo:lo + _B]
            for j in range(8):
                i = c * 8 + j
                row = rows8[j:j + 1, :]
                ki = jnp.sum(jnp.where(lidx == i, kb, 0.0))
                sup = (row > _NMS_THRESH) & (lidx > i) & (ki > 0.5)
                kb = jnp.where(sup, 0.0, kb)
            return kb

        kb = jax.lax.fori_loop(0, _B // 8, body, keep_ref[0:1, lo:lo + _B])
        keep_ref[0:1, lo:lo + _B] = kb

        row_gi = jax.lax.broadcasted_iota(jnp.int32, (_B, 1), 0) + lo
        s_mat = jnp.where((iou_ref[:, :] > _NMS_THRESH) & (col_i > row_gi),
                          1.0, 0.0)
        sup_cnt = jnp.dot(kb, s_mat, preferred_element_type=jnp.float32)
        keep_ref[0:1, :] = jnp.where(sup_cnt > 0.5, 0.0, keep_ref[0:1, :])

    keep = keep_ref[0:1, :]
    tri = jnp.where(
        jax.lax.broadcasted_iota(jnp.int32, (_K, _K), 0)
        <= jax.lax.broadcasted_iota(jnp.int32, (_K, _K), 1), 1.0, 0.0)
    cum = jnp.dot(keep, tri, preferred_element_type=jnp.float32)
    slot = cum.astype(jnp.int32) - 1
    s_iota = jax.lax.broadcasted_iota(jnp.int32, (128, 1), 0)
    onehot = jnp.where((s_iota == slot) & (keep > 0.5), 1.0, 0.0)
    out_ref[:, :] = jax.lax.dot_general(
        onehot, cols_ref[:, :], (((1,), (0,)), ((), ())),
        precision=jax.lax.Precision.HIGHEST,
        preferred_element_type=jnp.float32)


def kernel(boxes, scores):
    valid = scores > _PRE_NMS_THRESH
    masked = jnp.where(valid, scores, -1.0)
    top_scores, top_idx = jax.lax.top_k(masked, _PRE_NMS_TOPK)
    top_boxes = jnp.take(boxes, top_idx, axis=0)

    pad = _K - _PRE_NMS_TOPK
    ts = jnp.pad(top_scores, (0, pad), constant_values=-1.0)
    tb = jnp.pad(top_boxes, ((0, pad), (0, 0)))
    cols = jnp.concatenate(
        [tb, ts[:, None], jnp.zeros((_K, 3), jnp.float32)], axis=1)
    rows = cols.T

    out = pl.pallas_call(
        _nms_kernel,
        out_shape=jax.ShapeDtypeStruct((128, 8), jnp.float32),
        scratch_shapes=[
            pltpu.VMEM((_B, _K), jnp.float32),
            pltpu.VMEM((8, _K), jnp.float32),
        ],
    )(rows, cols)
    return out[:_POST_NMS_TOPK, :5]


# speedup vs baseline: 51.6825x; 3.1384x over previous
import jax
import jax.numpy as jnp
from jax.experimental import pallas as pl
from jax.experimental.pallas import tpu as pltpu

_PRE_NMS_THRESH = 0.05
_PRE_NMS_TOPK = 1000
_POST_NMS_TOPK = 100
_NMS_THRESH = 0.6

_K = 1024
_B = 128
_NB = _K // _B


def _nms_kernel(rows_ref, cols_ref, out_ref, iou_ref, keep_ref):
    x0r = rows_ref[0:1, :]
    y0r = rows_ref[1:2, :]
    x1r = rows_ref[2:3, :]
    y1r = rows_ref[3:4, :]
    sr = rows_ref[4:5, :]
    area_r = jnp.maximum(x1r - x0r, 0.0) * jnp.maximum(y1r - y0r, 0.0)

    col_i = jax.lax.broadcasted_iota(jnp.int32, (1, _K), 1)
    lidx = jax.lax.broadcasted_iota(jnp.int32, (1, _B), 1)

    keep_ref[0:1, :] = jnp.where(sr > _PRE_NMS_THRESH, 1.0, 0.0)

    for b in range(_NB):
        lo = b * _B

        def block(lo=lo):
            x0c = cols_ref[lo:lo + _B, 0:1]
            y0c = cols_ref[lo:lo + _B, 1:2]
            x1c = cols_ref[lo:lo + _B, 2:3]
            y1c = cols_ref[lo:lo + _B, 3:4]
            area_c = jnp.maximum(x1c - x0c, 0.0) * jnp.maximum(y1c - y0c, 0.0)
            w = jnp.maximum(jnp.minimum(x1c, x1r) - jnp.maximum(x0c, x0r), 0.0)
            h = jnp.maximum(jnp.minimum(y1c, y1r) - jnp.maximum(y0c, y0r), 0.0)
            inter = w * h
            union = area_c + area_r - inter
            iou_ref[:, :] = inter / jnp.maximum(union, 1e-6)

            def body(c, kb):
                rows8 = iou_ref[pl.ds(c * 8, 8), lo:lo + _B]
                for j in range(8):
                    i = c * 8 + j
                    row = rows8[j:j + 1, :]
                    ki = jnp.sum(jnp.where(lidx == i, kb, 0.0))
                    sup = (row > _NMS_THRESH) & (lidx > i) & (ki > 0.5)
                    kb = jnp.where(sup, 0.0, kb)
                return kb

            kb = jax.lax.fori_loop(0, _B // 8, body, keep_ref[0:1, lo:lo + _B])
            keep_ref[0:1, lo:lo + _B] = kb

            row_gi = jax.lax.broadcasted_iota(jnp.int32, (_B, 1), 0) + lo
            s_mat = jnp.where((iou_ref[:, :] > _NMS_THRESH) & (col_i > row_gi),
                              1.0, 0.0)
            sup_cnt = jnp.dot(kb, s_mat, preferred_element_type=jnp.float32)
            keep_ref[0:1, :] = jnp.where(sup_cnt > 0.5, 0.0, keep_ref[0:1, :])

        if b == 0:
            block()
        else:
            done = jnp.sum(
                jnp.where(col_i < lo, keep_ref[0:1, :], 0.0))
            pl.when(done < float(_POST_NMS_TOPK))(block)

    keep = keep_ref[0:1, :]
    tri = jnp.where(
        jax.lax.broadcasted_iota(jnp.int32, (_K, _K), 0)
        <= jax.lax.broadcasted_iota(jnp.int32, (_K, _K), 1), 1.0, 0.0)
    cum = jnp.dot(keep, tri, preferred_element_type=jnp.float32)
    slot = cum.astype(jnp.int32) - 1
    s_iota = jax.lax.broadcasted_iota(jnp.int32, (128, 1), 0)
    onehot = jnp.where((s_iota == slot) & (keep > 0.5), 1.0, 0.0)
    out_ref[:, :] = jax.lax.dot_general(
        onehot, cols_ref[:, :], (((1,), (0,)), ((), ())),
        precision=jax.lax.Precision.HIGHEST,
        preferred_element_type=jnp.float32)


def kernel(boxes, scores):
    valid = scores > _PRE_NMS_THRESH
    masked = jnp.where(valid, scores, -1.0)
    top_scores, top_idx = jax.lax.top_k(masked, _PRE_NMS_TOPK)
    top_boxes = jnp.take(boxes, top_idx, axis=0)

    pad = _K - _PRE_NMS_TOPK
    ts = jnp.pad(top_scores, (0, pad), constant_values=-1.0)
    tb = jnp.pad(top_boxes, ((0, pad), (0, 0)))
    cols = jnp.concatenate(
        [tb, ts[:, None], jnp.zeros((_K, 3), jnp.float32)], axis=1)
    rows = cols.T

    out = pl.pallas_call(
        _nms_kernel,
        out_shape=jax.ShapeDtypeStruct((128, 8), jnp.float32),
        scratch_shapes=[
            pltpu.VMEM((_B, _K), jnp.float32),
            pltpu.VMEM((8, _K), jnp.float32),
        ],
    )(rows, cols)
    return out[:_POST_NMS_TOPK, :5]
